# baseline (speedup 1.0000x reference)
import functools

import jax
import jax.numpy as jnp
from jax import lax
from jax.experimental import pallas as pl
from jax.experimental.pallas import tpu as pltpu
from jax.experimental.pallas import tpu_sc as plsc

D_SPECIES = 32
D_TYPE = 8
D_ITEM = 16
D_ABILITY = 16
D_MOVE = 32
D_SCAL = 223
D_OUT = 439

C = 128

_FIELDS = (
    (0, D_SPECIES),
    (32, D_TYPE),
    (40, D_TYPE),
    (48, D_TYPE),
    (56, D_ITEM),
    (72, D_ABILITY),
    (88, D_MOVE),
    (120, D_MOVE),
    (152, D_MOVE),
    (184, D_MOVE),
)


def _make_kernel(rows: int):
    info = plsc.get_sparse_core_info()
    nw = info.num_cores * info.num_subcores
    assert rows % (nw * C) == 0
    rows_per_w = rows // nw
    n_chunks = rows_per_w // C

    mesh = plsc.VectorSubcoreMesh(core_axis_name="c", subcore_axis_name="s")

    @functools.partial(
        pl.kernel,
        out_type=jax.ShapeDtypeStruct((rows, D_OUT), jnp.float32),
        mesh=mesh,
        scratch_types=[
            pltpu.VMEM((10, C), jnp.int32),
            [pltpu.VMEM((C, w), jnp.float32) for _, w in _FIELDS],
            pltpu.SemaphoreType.DMA,
            pltpu.SemaphoreType.DMA,
        ],
        compiler_params=pltpu.CompilerParams(use_tc_tiling_on_sc=False),
    )
    def k(sp_i, t1_i, t2_i, tr_i, it_i, ab_i, mv0_i, mv1_i, mv2_i, mv3_i,
          scal, sp_t, ty_t, it_t, ab_t, mv_t, out, idx_v, bufs, gsem, wsem):
        wid = lax.axis_index("s") * info.num_cores + lax.axis_index("c")
        w_base = wid * rows_per_w
        idx_in = (sp_i, t1_i, t2_i, tr_i, it_i, ab_i, mv0_i, mv1_i, mv2_i,
                  mv3_i)
        tabs = (sp_t, ty_t, ty_t, ty_t, it_t, ab_t, mv_t, mv_t, mv_t, mv_t)

        def chunk(ci, carry):
            base = w_base + ci * C
            scp = pltpu.async_copy(
                scal.at[pl.ds(base, C)],
                out.at[pl.ds(base, C), pl.ds(216, D_SCAL)], wsem)
            for f in range(10):
                pltpu.sync_copy(idx_in[f].at[pl.ds(base, C)], idx_v.at[f])
            gcps = [
                pltpu.async_copy(tabs[f].at[idx_v.at[f]], bufs[f], gsem)
                for f in range(10)
            ]
            wcps = []
            for f, (off, w) in enumerate(_FIELDS):
                gcps[f].wait()
                wcps.append(pltpu.async_copy(
                    bufs[f], out.at[pl.ds(base, C), pl.ds(off, w)], wsem))
            for cp in wcps:
                cp.wait()
            scp.wait()
            return carry

        lax.fori_loop(0, n_chunks, chunk, 0)

    return k


def kernel(species_idx, type1_idx, type2_idx, tera_idx, item_idx, ability_idx,
           move_idx, scalars, species_embed, type_embed, item_embed,
           ability_embed, move_embed):
    b, n = species_idx.shape
    rows = b * n
    flat = lambda a: a.reshape(rows).astype(jnp.int32)
    mv = move_idx.reshape(rows, 4).astype(jnp.int32)
    k = _make_kernel(rows)
    out = k(flat(species_idx), flat(type1_idx), flat(type2_idx),
            flat(tera_idx), flat(item_idx), flat(ability_idx),
            mv[:, 0], mv[:, 1], mv[:, 2], mv[:, 3],
            scalars.reshape(rows, D_SCAL),
            species_embed, type_embed, item_embed, ability_embed, move_embed)
    return out.reshape(b, n, D_OUT)
